# baseline (speedup 1.0000x reference)
import functools

import jax
import jax.numpy as jnp
import numpy as np
from jax import lax
from jax.experimental import pallas as pl
from jax.experimental.pallas import tpu as pltpu

B = 64
S = 2048
D = 128
K = 15
NC = 256
N = S // 4


def _shift_down(z, o, rows):
    return jnp.concatenate([jnp.zeros((o, z.shape[1]), z.dtype), z[: rows - o]], axis=0)


def _shift_up(z, o, rows):
    return jnp.concatenate([z[o:], jnp.zeros((o, z.shape[1]), z.dtype)], axis=0)


def _dot(a, b):
    return jnp.dot(a, b, preferred_element_type=jnp.float32)


def _gelu(x):
    return 0.5 * x * (1.0 + lax.erf(x * 0.7071067811865475))


def _fwd(x4_ref, a6_ref, ms_ref, c0b_ref, c1w_ref, c1b_ref,
         gw0_ref, gb0_ref, gw1_ref, gb1_ref, gw2_ref, gb2_ref, gw3_ref, gb3_ref,
         aw_ref, lg_ref, lb_ref, fw_ref, fb_ref, o_ref):
    x4 = x4_ref[0]
    x3m = _shift_down(x4[:, 3:4], 1, N)
    x0p = _shift_up(x4[:, 0:1], 1, N)
    xc = jnp.concatenate([x4, x3m, x0p], axis=1)

    b0 = c0b_ref[:]
    pe = (jnp.maximum(_dot(xc, a6_ref[:, :D]) + b0, 0.0)
          + jnp.maximum(_dot(xc, a6_ref[:, D:2 * D]) + b0, 0.0))
    po = (jnp.maximum(_dot(xc, a6_ref[:, 2 * D:3 * D]) + b0, 0.0)
          + jnp.maximum(_dot(xc, a6_ref[:, 3 * D:]) + b0, 0.0))

    b1 = c1b_ref[:]
    pom = _shift_down(po, 1, N)
    pep = _shift_up(pe, 1, N)
    ye = jnp.maximum(_dot(pom, c1w_ref[0]) + _dot(pe, c1w_ref[1])
                     + _dot(po, c1w_ref[2]) + b1, 0.0)
    yo = jnp.maximum(_dot(pe, c1w_ref[0]) + _dot(po, c1w_ref[1])
                     + _dot(pep, c1w_ref[2]) + b1, 0.0)
    xg = ye + yo

    E = 16
    cK = float(1.0 / np.sqrt(2 * K + 1))
    ii = lax.broadcasted_iota(jnp.int32, (E, D), 0).astype(jnp.float32)
    etop = lax.rsqrt(jnp.minimum(ii, float(K)) + float(K) + 1.0)
    ebot = lax.rsqrt(jnp.minimum(float(N - 1) - (float(N - E) + ii), float(K))
                     + float(K) + 1.0)

    def _dscale(v):
        return jnp.concatenate(
            [v[:E] * etop, v[E:N - E] * cK, v[N - E:] * ebot], axis=0)

    ms = ms_ref[:]
    zpad = jnp.zeros((E, D), jnp.float32)
    for w_ref, b_ref in ((gw0_ref, gb0_ref), (gw1_ref, gb1_ref),
                         (gw2_ref, gb2_ref), (gw3_ref, gb3_ref)):
        z = _dscale(xg)
        zp = jnp.concatenate([zpad, z, zpad], axis=0)
        s = jnp.concatenate(
            [_dot(ms, zp[blk * 128: blk * 128 + 160]) for blk in range(N // 128)],
            axis=0)
        h = _dscale(s)
        h = _gelu(_dot(h, w_ref[:]) + b_ref[:])
        xg = xg + h

    sb = _dot(xg, aw_ref[:])
    sb = sb - jnp.max(sb)
    eb = jnp.exp(sb)
    se = jnp.sum(eb, axis=0, keepdims=True)
    pooled = jnp.sum(eb * xg, axis=0, keepdims=True) / se

    mu = jnp.mean(pooled, axis=-1, keepdims=True)
    var = jnp.mean((pooled - mu) ** 2, axis=-1, keepdims=True)
    pooled = (pooled - mu) * lax.rsqrt(var + 1e-6) * lg_ref[:] + lb_ref[:]

    o_ref[0] = _dot(pooled, fw_ref[:]) + fb_ref[:]


def kernel(inputs, conv0_w, conv0_b, conv1_w, conv1_b,
           gcn_w0, gcn_b0, gcn_w1, gcn_b1, gcn_w2, gcn_b2, gcn_w3, gcn_b3,
           attn_w, attn_b, ln_g, ln_b, fc_w, fc_b):
    xr = inputs.reshape(B, N, 4)
    w0, w1, w2 = conv0_w[0, 0], conv0_w[1, 0], conv0_w[2, 0]
    zD = jnp.zeros((D,), jnp.float32)
    a6 = jnp.stack([
        jnp.concatenate([w1, w0, zD, zD]),
        jnp.concatenate([w2, w1, w0, zD]),
        jnp.concatenate([zD, w2, w1, w0]),
        jnp.concatenate([zD, zD, w2, w1]),
        jnp.concatenate([w0, zD, zD, zD]),
        jnp.concatenate([zD, zD, zD, w2]),
    ], axis=0)
    aw_t = jnp.tile(attn_w, (1, D))
    rr = jnp.arange(128)[:, None]
    qq = jnp.arange(160)[None, :]
    mstrip = ((qq - rr >= 1) & (qq - rr <= 31)).astype(jnp.float32)
    c0b = conv0_b.reshape(1, D)
    c1w = conv1_w * 0.25
    c1b = conv1_b.reshape(1, D) * 0.5
    gb0 = gcn_b0.reshape(1, D)
    gb1 = gcn_b1.reshape(1, D)
    gb2 = gcn_b2.reshape(1, D)
    gb3 = gcn_b3.reshape(1, D)
    del attn_b
    lg = ln_g.reshape(1, D)
    lb = ln_b.reshape(1, D)
    fb = fc_b.reshape(1, NC)

    def full(arr):
        nd = arr.ndim
        return pl.BlockSpec(arr.shape, lambda b: (0,) * nd)

    operands = (xr, a6, mstrip, c0b, c1w, c1b,
                gcn_w0, gb0, gcn_w1, gb1, gcn_w2, gb2, gcn_w3, gb3,
                aw_t, lg, lb, fc_w, fb)
    in_specs = [pl.BlockSpec((1, N, 4), lambda b: (b, 0, 0))]
    in_specs += [full(a) for a in operands[1:]]

    out = pl.pallas_call(
        _fwd,
        grid=(B,),
        in_specs=in_specs,
        out_specs=pl.BlockSpec((1, 1, NC), lambda b: (b, 0, 0)),
        out_shape=jax.ShapeDtypeStruct((B, 1, NC), jnp.float32),
        compiler_params=pltpu.CompilerParams(
            dimension_semantics=("parallel",),
        ),
    )(*operands)
    return (out[:, 0, :],)


# speedup vs baseline: 1.5486x; 1.5486x over previous
import functools

import jax
import jax.numpy as jnp
import numpy as np
from jax import lax
from jax.experimental import pallas as pl
from jax.experimental.pallas import tpu as pltpu

B = 64
S = 2048
D = 128
K = 15
NC = 256
N = S // 4


def _shift_down(z, o, rows):
    return jnp.concatenate([jnp.zeros((o, z.shape[1]), z.dtype), z[: rows - o]], axis=0)


def _shift_up(z, o, rows):
    return jnp.concatenate([z[o:], jnp.zeros((o, z.shape[1]), z.dtype)], axis=0)


def _dot(a, b):
    return jnp.dot(a, b, preferred_element_type=jnp.float32)


def _gelu(x):
    return 0.5 * x * (1.0 + lax.erf(x * 0.7071067811865475))


def _fwd(x4_ref, a6_ref, ms_ref, c0b_ref, c1w_ref, c1b_ref,
         gw0_ref, gb0_ref, gw1_ref, gb1_ref, gw2_ref, gb2_ref, gw3_ref, gb3_ref,
         aw_ref, lg_ref, lb_ref, fw_ref, fb_ref, o_ref):
    b0 = c0b_ref[:]
    b1 = c1b_ref[:]

    def front(el):
        x4 = x4_ref[el]
        x3m = _shift_down(x4[:, 3:4], 1, N)
        x0p = _shift_up(x4[:, 0:1], 1, N)
        xc = jnp.concatenate([x4, x3m, x0p], axis=1)
        pe = (jnp.maximum(_dot(xc, a6_ref[:, :D]) + b0, 0.0)
              + jnp.maximum(_dot(xc, a6_ref[:, D:2 * D]) + b0, 0.0))
        po = (jnp.maximum(_dot(xc, a6_ref[:, 2 * D:3 * D]) + b0, 0.0)
              + jnp.maximum(_dot(xc, a6_ref[:, 3 * D:]) + b0, 0.0))
        pom = _shift_down(po, 1, N)
        pep = _shift_up(pe, 1, N)
        ye = jnp.maximum(_dot(pom, c1w_ref[0]) + _dot(pe, c1w_ref[1])
                         + _dot(po, c1w_ref[2]) + b1, 0.0)
        yo = jnp.maximum(_dot(pe, c1w_ref[0]) + _dot(po, c1w_ref[1])
                         + _dot(pep, c1w_ref[2]) + b1, 0.0)
        return ye + yo

    xg = jnp.concatenate([front(0), front(1)], axis=1)
    D2 = 2 * D

    E = 16
    cK = float(1.0 / np.sqrt(2 * K + 1))
    ii = lax.broadcasted_iota(jnp.int32, (E, D2), 0).astype(jnp.float32)
    etop = lax.rsqrt(jnp.minimum(ii, float(K)) + float(K) + 1.0)
    ebot = lax.rsqrt(jnp.minimum(float(N - 1) - (float(N - E) + ii), float(K))
                     + float(K) + 1.0)

    def _dscale(v):
        return jnp.concatenate(
            [v[:E] * etop, v[E:N - E] * cK, v[N - E:] * ebot], axis=0)

    ms = ms_ref[:]
    zpad = jnp.zeros((E, D2), jnp.float32)
    for w_ref, b_ref in ((gw0_ref, gb0_ref), (gw1_ref, gb1_ref),
                         (gw2_ref, gb2_ref), (gw3_ref, gb3_ref)):
        z = _dscale(xg)
        zp = jnp.concatenate([zpad, z, zpad], axis=0)
        s = jnp.concatenate(
            [_dot(ms, zp[blk * 128: blk * 128 + 160]) for blk in range(N // 128)],
            axis=0)
        h = _dscale(s)
        w = w_ref[:]
        bb = b_ref[:]
        h = _gelu(jnp.concatenate(
            [_dot(h[:, :D], w) + bb, _dot(h[:, D:], w) + bb], axis=1))
        xg = xg + h

    aw = aw_ref[:]
    sb = jnp.concatenate([_dot(xg[:, :D], aw), _dot(xg[:, D:], aw)], axis=1)
    sb = sb - jnp.max(sb, axis=0, keepdims=True)
    eb = jnp.exp(sb)
    se = jnp.sum(eb, axis=0, keepdims=True)
    pooled = jnp.sum(eb * xg, axis=0, keepdims=True) / se

    for el in (0, 1):
        p1 = pooled[:, el * D:(el + 1) * D]
        mu = jnp.mean(p1, axis=-1, keepdims=True)
        var = jnp.mean((p1 - mu) ** 2, axis=-1, keepdims=True)
        p1 = (p1 - mu) * lax.rsqrt(var + 1e-6) * lg_ref[:] + lb_ref[:]
        o_ref[el] = _dot(p1, fw_ref[:]) + fb_ref[:]


def kernel(inputs, conv0_w, conv0_b, conv1_w, conv1_b,
           gcn_w0, gcn_b0, gcn_w1, gcn_b1, gcn_w2, gcn_b2, gcn_w3, gcn_b3,
           attn_w, attn_b, ln_g, ln_b, fc_w, fc_b):
    xr = inputs.reshape(B, N, 4)
    w0, w1, w2 = conv0_w[0, 0], conv0_w[1, 0], conv0_w[2, 0]
    zD = jnp.zeros((D,), jnp.float32)
    a6 = jnp.stack([
        jnp.concatenate([w1, w0, zD, zD]),
        jnp.concatenate([w2, w1, w0, zD]),
        jnp.concatenate([zD, w2, w1, w0]),
        jnp.concatenate([zD, zD, w2, w1]),
        jnp.concatenate([w0, zD, zD, zD]),
        jnp.concatenate([zD, zD, zD, w2]),
    ], axis=0)
    aw_t = jnp.tile(attn_w, (1, D))
    rr = jnp.arange(128)[:, None]
    qq = jnp.arange(160)[None, :]
    mstrip = ((qq - rr >= 1) & (qq - rr <= 31)).astype(jnp.float32)
    c0b = conv0_b.reshape(1, D)
    c1w = conv1_w * 0.25
    c1b = conv1_b.reshape(1, D) * 0.5
    gb0 = gcn_b0.reshape(1, D)
    gb1 = gcn_b1.reshape(1, D)
    gb2 = gcn_b2.reshape(1, D)
    gb3 = gcn_b3.reshape(1, D)
    del attn_b
    lg = ln_g.reshape(1, D)
    lb = ln_b.reshape(1, D)
    fb = fc_b.reshape(1, NC)

    def full(arr):
        nd = arr.ndim
        return pl.BlockSpec(arr.shape, lambda b: (0,) * nd)

    operands = (xr, a6, mstrip, c0b, c1w, c1b,
                gcn_w0, gb0, gcn_w1, gb1, gcn_w2, gb2, gcn_w3, gb3,
                aw_t, lg, lb, fc_w, fb)
    in_specs = [pl.BlockSpec((2, N, 4), lambda b: (b, 0, 0))]
    in_specs += [full(a) for a in operands[1:]]

    out = pl.pallas_call(
        _fwd,
        grid=(B // 2,),
        in_specs=in_specs,
        out_specs=pl.BlockSpec((2, 1, NC), lambda b: (b, 0, 0)),
        out_shape=jax.ShapeDtypeStruct((B, 1, NC), jnp.float32),
        compiler_params=pltpu.CompilerParams(
            dimension_semantics=("parallel",),
        ),
    )(*operands)
    return (out[:, 0, :],)


# speedup vs baseline: 1.9926x; 1.2867x over previous
import functools

import jax
import jax.numpy as jnp
import numpy as np
from jax import lax
from jax.experimental import pallas as pl
from jax.experimental.pallas import tpu as pltpu

B = 64
S = 2048
D = 128
K = 15
NC = 256
N = S // 4
EL = 4


def _shift_down(z, o, rows):
    return jnp.concatenate([jnp.zeros((o, z.shape[1]), z.dtype), z[: rows - o]], axis=0)


def _shift_up(z, o, rows):
    return jnp.concatenate([z[o:], jnp.zeros((o, z.shape[1]), z.dtype)], axis=0)


def _dot(a, b):
    return jnp.dot(a, b, preferred_element_type=jnp.float32)


def _gelu(x):
    return 0.5 * x * (1.0 + lax.erf(x * 0.7071067811865475))


def _fwd(x4_ref, a6_ref, ms_ref, c0b_ref, c1w_ref, c1b_ref,
         gw0_ref, gb0_ref, gw1_ref, gb1_ref, gw2_ref, gb2_ref, gw3_ref, gb3_ref,
         aw_ref, lg_ref, lb_ref, fw_ref, fb_ref, o_ref):
    b0 = c0b_ref[:]
    b1 = c1b_ref[:]

    def front(el):
        x4 = x4_ref[el]
        x3m = _shift_down(x4[:, 3:4], 1, N)
        x0p = _shift_up(x4[:, 0:1], 1, N)
        xc = jnp.concatenate([x4, x3m, x0p], axis=1)
        pe = (jnp.maximum(_dot(xc, a6_ref[:, :D]) + b0, 0.0)
              + jnp.maximum(_dot(xc, a6_ref[:, D:2 * D]) + b0, 0.0))
        po = (jnp.maximum(_dot(xc, a6_ref[:, 2 * D:3 * D]) + b0, 0.0)
              + jnp.maximum(_dot(xc, a6_ref[:, 3 * D:]) + b0, 0.0))
        pom = _shift_down(po, 1, N)
        pep = _shift_up(pe, 1, N)
        ye = jnp.maximum(_dot(pom, c1w_ref[0]) + _dot(pe, c1w_ref[1])
                         + _dot(po, c1w_ref[2]) + b1, 0.0)
        yo = jnp.maximum(_dot(pe, c1w_ref[0]) + _dot(po, c1w_ref[1])
                         + _dot(pep, c1w_ref[2]) + b1, 0.0)
        return ye + yo

    xg = jnp.concatenate([front(el) for el in range(EL)], axis=1)
    D2 = EL * D

    E = 16
    cK = float(1.0 / np.sqrt(2 * K + 1))
    ii = lax.broadcasted_iota(jnp.int32, (E, D2), 0).astype(jnp.float32)
    etop = lax.rsqrt(jnp.minimum(ii, float(K)) + float(K) + 1.0)
    ebot = lax.rsqrt(jnp.minimum(float(N - 1) - (float(N - E) + ii), float(K))
                     + float(K) + 1.0)

    def _dscale(v):
        return jnp.concatenate(
            [v[:E] * etop, v[E:N - E] * cK, v[N - E:] * ebot], axis=0)

    ms = ms_ref[:]
    zpad = jnp.zeros((E, D2), jnp.float32)
    for w_ref, b_ref in ((gw0_ref, gb0_ref), (gw1_ref, gb1_ref),
                         (gw2_ref, gb2_ref), (gw3_ref, gb3_ref)):
        z = _dscale(xg)
        zp = jnp.concatenate([zpad, z, zpad], axis=0)
        s = jnp.concatenate(
            [_dot(ms, zp[blk * 128: blk * 128 + 160]) for blk in range(N // 128)],
            axis=0)
        h = _dscale(s)
        w = w_ref[:]
        bb = b_ref[:]
        h = _gelu(jnp.concatenate(
            [_dot(h[:, el * D:(el + 1) * D], w) + bb for el in range(EL)],
            axis=1))
        xg = xg + h

    aw = aw_ref[:]
    sb = jnp.concatenate(
        [_dot(xg[:, el * D:(el + 1) * D], aw) for el in range(EL)], axis=1)
    sb = sb - jnp.max(sb, axis=0, keepdims=True)
    eb = jnp.exp(sb)
    se = jnp.sum(eb, axis=0, keepdims=True)
    pooled = jnp.sum(eb * xg, axis=0, keepdims=True) / se

    for el in range(EL):
        p1 = pooled[:, el * D:(el + 1) * D]
        mu = jnp.mean(p1, axis=-1, keepdims=True)
        var = jnp.mean((p1 - mu) ** 2, axis=-1, keepdims=True)
        p1 = (p1 - mu) * lax.rsqrt(var + 1e-6) * lg_ref[:] + lb_ref[:]
        o_ref[el] = _dot(p1, fw_ref[:]) + fb_ref[:]


def kernel(inputs, conv0_w, conv0_b, conv1_w, conv1_b,
           gcn_w0, gcn_b0, gcn_w1, gcn_b1, gcn_w2, gcn_b2, gcn_w3, gcn_b3,
           attn_w, attn_b, ln_g, ln_b, fc_w, fc_b):
    xr = inputs.reshape(B, N, 4)
    w0, w1, w2 = conv0_w[0, 0], conv0_w[1, 0], conv0_w[2, 0]
    zD = jnp.zeros((D,), jnp.float32)
    a6 = jnp.stack([
        jnp.concatenate([w1, w0, zD, zD]),
        jnp.concatenate([w2, w1, w0, zD]),
        jnp.concatenate([zD, w2, w1, w0]),
        jnp.concatenate([zD, zD, w2, w1]),
        jnp.concatenate([w0, zD, zD, zD]),
        jnp.concatenate([zD, zD, zD, w2]),
    ], axis=0)
    aw_t = jnp.tile(attn_w, (1, D))
    rr = jnp.arange(128)[:, None]
    qq = jnp.arange(160)[None, :]
    mstrip = ((qq - rr >= 1) & (qq - rr <= 31)).astype(jnp.float32)
    c0b = conv0_b.reshape(1, D)
    c1w = conv1_w * 0.25
    c1b = conv1_b.reshape(1, D) * 0.5
    gb0 = gcn_b0.reshape(1, D)
    gb1 = gcn_b1.reshape(1, D)
    gb2 = gcn_b2.reshape(1, D)
    gb3 = gcn_b3.reshape(1, D)
    del attn_b
    lg = ln_g.reshape(1, D)
    lb = ln_b.reshape(1, D)
    fb = fc_b.reshape(1, NC)

    def full(arr):
        nd = arr.ndim
        return pl.BlockSpec(arr.shape, lambda b: (0,) * nd)

    operands = (xr, a6, mstrip, c0b, c1w, c1b,
                gcn_w0, gb0, gcn_w1, gb1, gcn_w2, gb2, gcn_w3, gb3,
                aw_t, lg, lb, fc_w, fb)
    in_specs = [pl.BlockSpec((EL, N, 4), lambda b: (b, 0, 0))]
    in_specs += [full(a) for a in operands[1:]]

    out = pl.pallas_call(
        _fwd,
        grid=(B // EL,),
        in_specs=in_specs,
        out_specs=pl.BlockSpec((EL, 1, NC), lambda b: (b, 0, 0)),
        out_shape=jax.ShapeDtypeStruct((B, 1, NC), jnp.float32),
        compiler_params=pltpu.CompilerParams(
            dimension_semantics=("parallel",),
        ),
    )(*operands)
    return (out[:, 0, :],)


# speedup vs baseline: 2.1231x; 1.0655x over previous
import functools

import jax
import jax.numpy as jnp
import numpy as np
from jax import lax
from jax.experimental import pallas as pl
from jax.experimental.pallas import tpu as pltpu

B = 64
S = 2048
D = 128
K = 15
NC = 256
N = S // 4
EL = 8


def _shift_down(z, o, rows):
    return jnp.concatenate([jnp.zeros((o, z.shape[1]), z.dtype), z[: rows - o]], axis=0)


def _shift_up(z, o, rows):
    return jnp.concatenate([z[o:], jnp.zeros((o, z.shape[1]), z.dtype)], axis=0)


def _dot(a, b):
    return jnp.dot(a, b, preferred_element_type=jnp.float32)


def _gelu(x):
    return 0.5 * x * (1.0 + lax.erf(x * 0.7071067811865475))


def _fwd(x4_ref, a6_ref, ms_ref, c0b_ref, c1w_ref, c1b_ref,
         gw0_ref, gb0_ref, gw1_ref, gb1_ref, gw2_ref, gb2_ref, gw3_ref, gb3_ref,
         aw_ref, lg_ref, lb_ref, fw_ref, fb_ref, o_ref):
    b0 = c0b_ref[:]
    b1 = c1b_ref[:]

    def front(el):
        x4 = x4_ref[el]
        x3m = _shift_down(x4[:, 3:4], 1, N)
        x0p = _shift_up(x4[:, 0:1], 1, N)
        xc = jnp.concatenate([x4, x3m, x0p], axis=1)
        pe = (jnp.maximum(_dot(xc, a6_ref[:, :D]) + b0, 0.0)
              + jnp.maximum(_dot(xc, a6_ref[:, D:2 * D]) + b0, 0.0))
        po = (jnp.maximum(_dot(xc, a6_ref[:, 2 * D:3 * D]) + b0, 0.0)
              + jnp.maximum(_dot(xc, a6_ref[:, 3 * D:]) + b0, 0.0))
        pom = _shift_down(po, 1, N)
        pep = _shift_up(pe, 1, N)
        ye = jnp.maximum(_dot(pom, c1w_ref[0]) + _dot(pe, c1w_ref[1])
                         + _dot(po, c1w_ref[2]) + b1, 0.0)
        yo = jnp.maximum(_dot(pe, c1w_ref[0]) + _dot(po, c1w_ref[1])
                         + _dot(pep, c1w_ref[2]) + b1, 0.0)
        return ye + yo

    xg = jnp.concatenate([front(el) for el in range(EL)], axis=1)
    D2 = EL * D

    E = 16
    cK = float(1.0 / np.sqrt(2 * K + 1))
    ii = lax.broadcasted_iota(jnp.int32, (E, D2), 0).astype(jnp.float32)
    etop = lax.rsqrt(jnp.minimum(ii, float(K)) + float(K) + 1.0)
    ebot = lax.rsqrt(jnp.minimum(float(N - 1) - (float(N - E) + ii), float(K))
                     + float(K) + 1.0)

    def _dscale(v):
        return jnp.concatenate(
            [v[:E] * etop, v[E:N - E] * cK, v[N - E:] * ebot], axis=0)

    ms = ms_ref[:]
    zpad = jnp.zeros((E, D2), jnp.float32)
    for w_ref, b_ref in ((gw0_ref, gb0_ref), (gw1_ref, gb1_ref),
                         (gw2_ref, gb2_ref), (gw3_ref, gb3_ref)):
        z = _dscale(xg)
        zp = jnp.concatenate([zpad, z, zpad], axis=0)
        s = jnp.concatenate(
            [_dot(ms, zp[blk * 128: blk * 128 + 160]) for blk in range(N // 128)],
            axis=0)
        h = _dscale(s)
        w = w_ref[:]
        bb = b_ref[:]
        h = _gelu(jnp.concatenate(
            [_dot(h[:, el * D:(el + 1) * D], w) + bb for el in range(EL)],
            axis=1))
        xg = xg + h

    aw = aw_ref[:]
    sb = jnp.concatenate(
        [_dot(xg[:, el * D:(el + 1) * D], aw) for el in range(EL)], axis=1)
    sb = sb - jnp.max(sb, axis=0, keepdims=True)
    eb = jnp.exp(sb)
    se = jnp.sum(eb, axis=0, keepdims=True)
    pooled = jnp.sum(eb * xg, axis=0, keepdims=True) / se

    for el in range(EL):
        p1 = pooled[:, el * D:(el + 1) * D]
        mu = jnp.mean(p1, axis=-1, keepdims=True)
        var = jnp.mean((p1 - mu) ** 2, axis=-1, keepdims=True)
        p1 = (p1 - mu) * lax.rsqrt(var + 1e-6) * lg_ref[:] + lb_ref[:]
        o_ref[el] = _dot(p1, fw_ref[:]) + fb_ref[:]


def kernel(inputs, conv0_w, conv0_b, conv1_w, conv1_b,
           gcn_w0, gcn_b0, gcn_w1, gcn_b1, gcn_w2, gcn_b2, gcn_w3, gcn_b3,
           attn_w, attn_b, ln_g, ln_b, fc_w, fc_b):
    xr = inputs.reshape(B, N, 4)
    w0, w1, w2 = conv0_w[0, 0], conv0_w[1, 0], conv0_w[2, 0]
    zD = jnp.zeros((D,), jnp.float32)
    a6 = jnp.stack([
        jnp.concatenate([w1, w0, zD, zD]),
        jnp.concatenate([w2, w1, w0, zD]),
        jnp.concatenate([zD, w2, w1, w0]),
        jnp.concatenate([zD, zD, w2, w1]),
        jnp.concatenate([w0, zD, zD, zD]),
        jnp.concatenate([zD, zD, zD, w2]),
    ], axis=0)
    aw_t = jnp.tile(attn_w, (1, D))
    rr = jnp.arange(128)[:, None]
    qq = jnp.arange(160)[None, :]
    mstrip = ((qq - rr >= 1) & (qq - rr <= 31)).astype(jnp.float32)
    c0b = conv0_b.reshape(1, D)
    c1w = conv1_w * 0.25
    c1b = conv1_b.reshape(1, D) * 0.5
    gb0 = gcn_b0.reshape(1, D)
    gb1 = gcn_b1.reshape(1, D)
    gb2 = gcn_b2.reshape(1, D)
    gb3 = gcn_b3.reshape(1, D)
    del attn_b
    lg = ln_g.reshape(1, D)
    lb = ln_b.reshape(1, D)
    fb = fc_b.reshape(1, NC)

    def full(arr):
        nd = arr.ndim
        return pl.BlockSpec(arr.shape, lambda b: (0,) * nd)

    operands = (xr, a6, mstrip, c0b, c1w, c1b,
                gcn_w0, gb0, gcn_w1, gb1, gcn_w2, gb2, gcn_w3, gb3,
                aw_t, lg, lb, fc_w, fb)
    in_specs = [pl.BlockSpec((EL, N, 4), lambda b: (b, 0, 0))]
    in_specs += [full(a) for a in operands[1:]]

    out = pl.pallas_call(
        _fwd,
        grid=(B // EL,),
        in_specs=in_specs,
        out_specs=pl.BlockSpec((EL, 1, NC), lambda b: (b, 0, 0)),
        out_shape=jax.ShapeDtypeStruct((B, 1, NC), jnp.float32),
        compiler_params=pltpu.CompilerParams(
            dimension_semantics=("parallel",),
        ),
    )(*operands)
    return (out[:, 0, :],)
